# baseline (speedup 1.0000x reference)
import jax
import jax.numpy as jnp
from jax.experimental import pallas as pl
from jax.experimental.pallas import tpu as pltpu

_C = 19
_EPS = 1e-06


def _dice_body(tgt_ref, xa_ref, xb_ref, out_ref, s_acc, i_acc, f_acc):
    n = pl.program_id(0)
    h = pl.program_id(1)
    num_n = pl.num_programs(0)
    num_h = pl.num_programs(1)

    ta = tgt_ref[0, 0]
    tb = tgt_ref[0, 1]
    first = (n == 0) & (h == 0)
    for c in range(_C):
        va = xa_ref[0, c, 0]
        vb = xb_ref[0, c, 0]
        eqa = ta == c
        eqb = tb == c
        psum = jnp.sum(va) + jnp.sum(vb)
        inter = jnp.sum(jnp.where(eqa, va, 0.0)) + jnp.sum(jnp.where(eqb, vb, 0.0))
        freq = jnp.sum(jnp.where(eqa, 1.0, 0.0)) + jnp.sum(jnp.where(eqb, 1.0, 0.0))

        @pl.when(first)
        def _init(c=c, psum=psum, inter=inter, freq=freq):
            s_acc[c] = psum
            i_acc[c] = inter
            f_acc[c] = freq

        @pl.when(jnp.logical_not(first))
        def _accum(c=c, psum=psum, inter=inter, freq=freq):
            s_acc[c] = s_acc[c] + psum
            i_acc[c] = i_acc[c] + inter
            f_acc[c] = f_acc[c] + freq

    @pl.when((n == num_n - 1) & (h == num_h - 1))
    def _finish():
        def tot_body(k, acc):
            return acc + f_acc[k]
        tot_f = jax.lax.fori_loop(0, _C, tot_body, 0.0)

        def loss_body(k, acc):
            fk = f_acc[k]
            ik = i_acc[k]
            uk = s_acc[k] + fk - ik
            dice = 1.0 - (2.0 * ik + _EPS) / (uk + _EPS)
            w = tot_f / (fk * _C)
            return acc + dice * w
        out_ref[0, 0] = jax.lax.fori_loop(0, _C, loss_body, 0.0)


def kernel(inputs, targets):
    N, C, H, W = inputs.shape
    HQ = H // 4
    t4 = targets.reshape(N, 4, HQ, W)
    x5 = inputs.reshape(N, C, 4, HQ, W)
    out = pl.pallas_call(
        _dice_body,
        grid=(N, 2),
        in_specs=[
            pl.BlockSpec((1, 2, HQ, W), lambda n, h: (n, h, 0, 0)),
            pl.BlockSpec((1, C, 1, HQ, W), lambda n, h: (n, 0, 2 * h, 0, 0)),
            pl.BlockSpec((1, C, 1, HQ, W), lambda n, h: (n, 0, 2 * h + 1, 0, 0)),
        ],
        out_specs=pl.BlockSpec(memory_space=pltpu.SMEM),
        out_shape=jax.ShapeDtypeStruct((1, 1), jnp.float32),
        scratch_shapes=[
            pltpu.SMEM((_C,), jnp.float32),
            pltpu.SMEM((_C,), jnp.float32),
            pltpu.SMEM((_C,), jnp.float32),
        ],
    )(t4, x5, x5)
    return out[0, 0]


# speedup vs baseline: 1.0066x; 1.0066x over previous
import jax
import jax.numpy as jnp
from jax.experimental import pallas as pl
from jax.experimental.pallas import tpu as pltpu

_C = 19
_EPS = 1e-06


def _dice_body(tgt_ref, x_ref, out_ref, s_acc, i_acc, f_acc):
    n = pl.program_id(0)
    h = pl.program_id(1)
    num_n = pl.num_programs(0)
    num_h = pl.num_programs(1)

    t = tgt_ref[0]
    first = (n == 0) & (h == 0)
    for c in range(_C):
        v = x_ref[0, c]
        eq = t == c
        psum = jnp.sum(v)
        inter = jnp.sum(jnp.where(eq, v, 0.0))
        freq = jnp.sum(jnp.where(eq, 1.0, 0.0))

        @pl.when(first)
        def _init(c=c, psum=psum, inter=inter, freq=freq):
            s_acc[c] = psum
            i_acc[c] = inter
            f_acc[c] = freq

        @pl.when(jnp.logical_not(first))
        def _accum(c=c, psum=psum, inter=inter, freq=freq):
            s_acc[c] = s_acc[c] + psum
            i_acc[c] = i_acc[c] + inter
            f_acc[c] = f_acc[c] + freq

    @pl.when((n == num_n - 1) & (h == num_h - 1))
    def _finish():
        def tot_body(k, acc):
            return acc + f_acc[k]
        tot_f = jax.lax.fori_loop(0, _C, tot_body, 0.0)

        def loss_body(k, acc):
            fk = f_acc[k]
            ik = i_acc[k]
            uk = s_acc[k] + fk - ik
            dice = 1.0 - (2.0 * ik + _EPS) / (uk + _EPS)
            w = tot_f / (fk * _C)
            return acc + dice * w
        out_ref[0, 0] = jax.lax.fori_loop(0, _C, loss_body, 0.0)


def kernel(inputs, targets):
    N, C, H, W = inputs.shape
    HB = H // 2
    out = pl.pallas_call(
        _dice_body,
        grid=(N, 2),
        in_specs=[
            pl.BlockSpec((1, HB, W), lambda n, h: (n, h, 0)),
            pl.BlockSpec((1, C, HB, W), lambda n, h: (n, 0, h, 0)),
        ],
        out_specs=pl.BlockSpec(memory_space=pltpu.SMEM),
        out_shape=jax.ShapeDtypeStruct((1, 1), jnp.float32),
        scratch_shapes=[
            pltpu.SMEM((_C,), jnp.float32),
            pltpu.SMEM((_C,), jnp.float32),
            pltpu.SMEM((_C,), jnp.float32),
        ],
    )(targets, inputs)
    return out[0, 0]
